# baseline (device time: 124267 ns/iter reference)
import jax
import jax.numpy as jnp
from jax import lax
from jax.experimental import pallas as pl
from jax.experimental.pallas import tpu as pltpu

M = 4096
N = 1024
H = M // 2
C = 32
Hc = H // C


def kernel(x):
    def body(x_ref, out_ref, recv1, p1_send, p1_recv, p2_send, p2_recv):
        my_x = lax.axis_index("x")
        my_y = lax.axis_index("y")

        barrier_sem = pltpu.get_barrier_semaphore()
        pl.semaphore_signal(
            barrier_sem, inc=1,
            device_id=(my_x, 1 - my_y), device_id_type=pl.DeviceIdType.MESH,
        )
        pl.semaphore_signal(
            barrier_sem, inc=1,
            device_id=(1 - my_x, my_y), device_id_type=pl.DeviceIdType.MESH,
        )
        pl.semaphore_wait(barrier_sem, 2)

        def run(mx):
            base = mx * H

            rdma1 = []
            for k in range(C):
                sl = pl.ds(k * Hc, Hc)
                r = pltpu.make_async_remote_copy(
                    src_ref=x_ref.at[pl.ds(base + k * Hc, Hc), :],
                    dst_ref=recv1.at[sl, :],
                    send_sem=p1_send.at[k],
                    recv_sem=p1_recv.at[k],
                    device_id=(mx, 1 - my_y),
                    device_id_type=pl.DeviceIdType.MESH,
                )
                r.start()
                rdma1.append(r)

            rdma2 = []
            for k in range(C):
                sl = pl.ds(k * Hc, Hc)
                out_sl = pl.ds(base + k * Hc, Hc)
                rdma1[k].wait_recv()
                recv1[sl, :] = recv1[sl, :] + x_ref[out_sl, :]
                r = pltpu.make_async_remote_copy(
                    src_ref=recv1.at[sl, :],
                    dst_ref=out_ref.at[out_sl, :],
                    send_sem=p2_send.at[k],
                    recv_sem=p2_recv.at[k],
                    device_id=(1 - mx, my_y),
                    device_id_type=pl.DeviceIdType.MESH,
                )
                r.start()
                rdma2.append(r)
                out_ref[out_sl, :] = recv1[sl, :]

            for k in range(C):
                rdma2[k].wait_recv()
            for k in range(C):
                rdma1[k].wait_send()
                rdma2[k].wait_send()

        @pl.when(my_x == 0)
        def _():
            run(0)

        @pl.when(my_x == 1)
        def _():
            run(1)

    return pl.pallas_call(
        body,
        out_shape=jax.ShapeDtypeStruct((M, N), jnp.float32),
        in_specs=[pl.BlockSpec(memory_space=pltpu.VMEM)],
        out_specs=pl.BlockSpec(memory_space=pltpu.VMEM),
        scratch_shapes=[
            pltpu.VMEM((H, N), jnp.float32),
            pltpu.SemaphoreType.DMA((C,)),
            pltpu.SemaphoreType.DMA((C,)),
            pltpu.SemaphoreType.DMA((C,)),
            pltpu.SemaphoreType.DMA((C,)),
        ],
        compiler_params=pltpu.CompilerParams(
            collective_id=0,
            vmem_limit_bytes=100 * 1024 * 1024,
        ),
    )(x)
